# baseline (speedup 1.0000x reference)
import functools

import jax
import jax.numpy as jnp
from jax import lax
from jax.experimental import pallas as pl
from jax.experimental.pallas import tpu as pltpu
from jax.experimental.pallas import tpu_sc as plsc

_NC = 2
_NS = 16
_NW = _NC * _NS
_L = 16


def _sc_interp(xt, gx, gy, *, img_h, img_w, n_chan, n_pts, pts_per_batch):
    n_words = xt.shape[1]
    cpts = 32
    nidx = 4 * cpts
    total_ch = n_pts // cpts
    chw, chr = divmod(total_ch, _NW)
    ppw_buf = (chw + 1) * cpts
    n_grp = ppw_buf // _L
    cvecs = n_chan // _L
    pix = img_h * img_w

    mesh = plsc.VectorSubcoreMesh(core_axis_name="c", subcore_axis_name="s",
                                  num_cores=_NC, num_subcores=_NS)

    @functools.partial(
        pl.kernel,
        out_type=jax.ShapeDtypeStruct((n_pts, n_chan), jnp.float32),
        mesh=mesh,
        scratch_types=[
            pltpu.VMEM((ppw_buf,), jnp.float32),
            pltpu.VMEM((ppw_buf,), jnp.float32),
            pltpu.VMEM((ppw_buf + _L,), jnp.float32),
            pltpu.VMEM((ppw_buf + _L,), jnp.float32),
            pltpu.VMEM((ppw_buf + _L,), jnp.float32),
            pltpu.VMEM((ppw_buf + _L,), jnp.float32),
            pltpu.VMEM((4 * ppw_buf,), jnp.int32),
            pltpu.VMEM((nidx, n_words), jnp.int32),
            pltpu.VMEM((nidx, n_words), jnp.int32),
            pltpu.VMEM((cpts, n_chan), jnp.float32),
            pltpu.SemaphoreType.DMA,
            pltpu.SemaphoreType.DMA,
        ],
    )
    def body(xt_hbm, gx_hbm, gy_hbm, out_hbm,
             gx_v, gy_v, wa_v, wb_v, wc_v, wd_v, idx_v, rows_v0, rows_v1,
             out_v, sem0, sem1):
        wid = lax.axis_index("s") * _NC + lax.axis_index("c")
        start_c = chw * wid + jnp.minimum(wid, chr)
        n_my = jnp.where(wid < chr, chw + 1, chw)
        p0 = start_c * cpts

        pltpu.sync_copy(gx_hbm.at[pl.ds(p0, ppw_buf)], gx_v)
        pltpu.sync_copy(gy_hbm.at[pl.ds(p0, ppw_buf)], gy_v)

        def wgt_body(g, _):
            sl = pl.ds(g * _L, _L)
            base_row = ((p0 + g * _L) // pts_per_batch) * pix
            xp = ((gx_v[sl] + 1.0) * img_w - 1.0) * 0.5
            yp = ((gy_v[sl] + 1.0) * img_h - 1.0) * 0.5
            xti = xp.astype(jnp.int32)
            x0 = jnp.where(xti.astype(jnp.float32) > xp, xti - 1, xti)
            yti = yp.astype(jnp.int32)
            y0 = jnp.where(yti.astype(jnp.float32) > yp, yti - 1, yti)
            x0f = x0.astype(jnp.float32)
            y0f = y0.astype(jnp.float32)
            ex = (x0f + 1.0) - xp
            ey = (y0f + 1.0) - yp
            sx = xp - x0f
            sy = yp - y0f
            zero = jnp.zeros_like(ex)
            vx0 = x0 >= 0
            vx1 = x0 <= img_w - 2
            vy0 = y0 >= 0
            vy1 = y0 <= img_h - 2
            wa = jnp.where(vx0 & vy0, ex * ey, zero)
            wb = jnp.where(vx0 & vy1, ex * sy, zero)
            wc = jnp.where(vx1 & vy0, sx * ey, zero)
            wd = jnp.where(vx1 & vy1, sx * sy, zero)
            x0c = jnp.maximum(x0, 0)
            x1c = jnp.minimum(x0 + 1, img_w - 1)
            y0c = jnp.maximum(y0, 0)
            y1c = jnp.minimum(y0 + 1, img_h - 1)
            ra = base_row + y0c * img_w + x0c
            rb = base_row + y1c * img_w + x0c
            rc = base_row + y0c * img_w + x1c
            rd = base_row + y1c * img_w + x1c
            wa_v[sl] = wa
            wb_v[sl] = wb
            wc_v[sl] = wc
            wd_v[sl] = wd
            ib = g * (4 * _L)
            idx_v[pl.ds(ib, _L)] = ra
            idx_v[pl.ds(ib + _L, _L)] = rb
            idx_v[pl.ds(ib + 2 * _L, _L)] = rc
            idx_v[pl.ds(ib + 3 * _L, _L)] = rd
            return 0

        lax.fori_loop(0, n_grp, wgt_body, 0)

        cmax = chw + 1

        def start_g(ch, rbuf, s):
            pltpu.async_copy(
                xt_hbm.at[idx_v.at[pl.ds(ch * nidx, nidx)]], rbuf, s)

        def wait_g(rbuf, s):
            pltpu.make_async_copy(
                xt_hbm.at[idx_v.at[pl.ds(0, nidx)]], rbuf, s).wait()

        lo_blk = n_words // _L
        hi_blk = (n_chan - n_words) // _L
        mask_hi = jnp.int32(-65536)

        def compute_out(ch, rbuf):
            def pt_body(j, _):
                p = ch * cpts + j
                was = wa_v[pl.ds(p, _L)][0]
                wbs = wb_v[pl.ds(p, _L)][0]
                wcs = wc_v[pl.ds(p, _L)][0]
                wds = wd_v[pl.ds(p, _L)][0]
                r0 = (4 * _L) * (j // _L) + (j % _L)
                for k in range(lo_blk):
                    cs = pl.ds(k * _L, _L)
                    va = rbuf[r0, cs]
                    vb = rbuf[r0 + _L, cs]
                    vc = rbuf[r0 + 2 * _L, cs]
                    vd = rbuf[r0 + 3 * _L, cs]
                    acc = (was * lax.bitcast_convert_type(va << 16, jnp.float32)
                           + wbs * lax.bitcast_convert_type(vb << 16, jnp.float32)
                           + wcs * lax.bitcast_convert_type(vc << 16, jnp.float32)
                           + wds * lax.bitcast_convert_type(vd << 16, jnp.float32))
                    out_v[j, cs] = acc
                    if k < hi_blk:
                        acch = (was * lax.bitcast_convert_type(va & mask_hi, jnp.float32)
                                + wbs * lax.bitcast_convert_type(vb & mask_hi, jnp.float32)
                                + wcs * lax.bitcast_convert_type(vc & mask_hi, jnp.float32)
                                + wds * lax.bitcast_convert_type(vd & mask_hi, jnp.float32))
                        out_v[j, pl.ds(n_words + k * _L, _L)] = acch
                return 0

            lax.fori_loop(0, cpts, pt_body, 0)

            @pl.when(ch < n_my)
            def _():
                pltpu.sync_copy(
                    out_v, out_hbm.at[pl.ds((start_c + ch) * cpts, cpts)])

        if cmax % 2 == 1:
            start_g(0, rows_v0, sem0)

            def pair_body(i, _):
                ch0 = 2 * i
                wait_g(rows_v0, sem0)
                start_g(ch0 + 1, rows_v1, sem1)
                compute_out(ch0, rows_v0)
                wait_g(rows_v1, sem1)
                start_g(ch0 + 2, rows_v0, sem0)
                compute_out(ch0 + 1, rows_v1)
                return 0

            lax.fori_loop(0, (cmax - 1) // 2, pair_body, 0)
            wait_g(rows_v0, sem0)
            compute_out(cmax - 1, rows_v0)
        else:
            def chunk_body(ch, _):
                start_g(ch, rows_v0, sem0)
                wait_g(rows_v0, sem0)
                compute_out(ch, rows_v0)
                return 0

            lax.fori_loop(0, cmax, chunk_body, 0)

    return body(xt, gx, gy)


def _tc_pack(x2):
    b, c, hw = x2.shape
    blk = 2048

    def rne(u):
        return (u + jnp.uint32(0x7FFF) + ((u >> 16) & jnp.uint32(1))) >> 16

    def body(lo_r, hi_r, out_r):
        lo = lax.bitcast_convert_type(lo_r[0], jnp.uint32)
        hi = lax.bitcast_convert_type(hi_r[0], jnp.uint32)
        lo16 = rne(lo)
        hi16 = rne(hi)
        z = jnp.zeros((256 - c, blk), jnp.uint32)
        hi16 = jnp.concatenate([hi16, z], axis=0)
        out_r[0] = lax.bitcast_convert_type(lo16 | (hi16 << 16), jnp.int32)

    return pl.pallas_call(
        body,
        grid=(b, hw // blk),
        in_specs=[
            pl.BlockSpec((1, 128, blk), lambda i, j: (i, 0, j)),
            pl.BlockSpec((1, c - 128, blk), lambda i, j: (i, 2, j)),
        ],
        out_specs=pl.BlockSpec((1, 128, blk), lambda i, j: (i, 0, j)),
        out_shape=jax.ShapeDtypeStruct((b, 128, hw), jnp.int32),
    )(x2, x2)


def _tc_transpose(x, c_pad):
    b, c, img_h, img_w = x.shape
    hw = img_h * img_w
    x2 = x.reshape(b, c, hw)
    blk = 512

    def body(xr, outr):
        t = jnp.swapaxes(xr[0], 0, 1)
        z = jnp.zeros((blk, c_pad - c), dtype=t.dtype)
        outr[0] = jnp.concatenate([t, z], axis=1)

    out = pl.pallas_call(
        body,
        grid=(b, hw // blk),
        in_specs=[pl.BlockSpec((1, c, blk), lambda i, j: (i, 0, j))],
        out_specs=pl.BlockSpec((1, blk, c_pad), lambda i, j: (i, j, 0)),
        out_shape=jax.ShapeDtypeStruct((b, hw, c_pad), jnp.float32),
    )(x2)
    return out.reshape(b * hw, c_pad)


def kernel(x, pos, H, W):
    b, c, img_h, img_w = x.shape
    n = pos.shape[1]
    hw = img_h * img_w
    words = _tc_pack(x.reshape(b, c, hw))
    xt = jnp.transpose(words, (0, 2, 1))
    xt = xt.reshape(b * hw, 128)
    scale = jnp.array([W - 1, H - 1], dtype=x.dtype)
    grid = 2.0 * (pos / scale) - 1.0
    n_pts = b * n
    cpts = 32
    total_ch = n_pts // cpts
    chw = total_ch // _NW
    pad = (chw + 1) * cpts + (total_ch - 1) * 0
    gx = jnp.pad(grid[..., 0].reshape(-1), (0, pad))
    gy = jnp.pad(grid[..., 1].reshape(-1), (0, pad))
    out = _sc_interp(xt, gx, gy, img_h=img_h, img_w=img_w, n_chan=c,
                     n_pts=n_pts, pts_per_batch=n)
    return out.reshape(b, n, c)


# speedup vs baseline: 1.4053x; 1.4053x over previous
import functools

import jax
import jax.numpy as jnp
from jax import lax
from jax.experimental import pallas as pl
from jax.experimental.pallas import tpu as pltpu
from jax.experimental.pallas import tpu_sc as plsc

_NC = 2
_NS = 16
_NW = _NC * _NS
_L = 16


def _sc_interp(xt, gx, gy, *, img_h, img_w, n_chan, n_pts, pts_per_batch):
    n_words = xt.shape[1]
    cpts = 32
    nidx = 4 * cpts
    total_ch = n_pts // cpts
    chw, chr = divmod(total_ch, _NW)
    ppw_buf = (chw + 1) * cpts
    n_grp = ppw_buf // _L
    cvecs = n_chan // _L
    pix = img_h * img_w

    mesh = plsc.VectorSubcoreMesh(core_axis_name="c", subcore_axis_name="s",
                                  num_cores=_NC, num_subcores=_NS)

    n_bat = n_pts // pts_per_batch

    @functools.partial(
        pl.kernel,
        out_type=jax.ShapeDtypeStruct((n_bat, pts_per_batch, n_chan),
                                      jnp.float32),
        mesh=mesh,
        scratch_types=[
            pltpu.VMEM((ppw_buf,), jnp.float32),
            pltpu.VMEM((ppw_buf,), jnp.float32),
            pltpu.VMEM((ppw_buf + _L,), jnp.float32),
            pltpu.VMEM((ppw_buf + _L,), jnp.float32),
            pltpu.VMEM((ppw_buf + _L,), jnp.float32),
            pltpu.VMEM((ppw_buf + _L,), jnp.float32),
            pltpu.VMEM((4 * ppw_buf,), jnp.int32),
            pltpu.VMEM((nidx, n_words), jnp.int32),
            pltpu.VMEM((nidx, n_words), jnp.int32),
            pltpu.VMEM((cpts, n_chan), jnp.float32),
            pltpu.SemaphoreType.DMA,
            pltpu.SemaphoreType.DMA,
        ],
    )
    def body(xt_hbm, gx_hbm, gy_hbm, out_hbm,
             gx_v, gy_v, wa_v, wb_v, wc_v, wd_v, idx_v, rows_v0, rows_v1,
             out_v, sem0, sem1):
        wid = lax.axis_index("s") * _NC + lax.axis_index("c")
        start_c = chw * wid + jnp.minimum(wid, chr)
        n_my = jnp.where(wid < chr, chw + 1, chw)
        p0 = start_c * cpts

        pltpu.sync_copy(gx_hbm.at[pl.ds(p0, ppw_buf)], gx_v)
        pltpu.sync_copy(gy_hbm.at[pl.ds(p0, ppw_buf)], gy_v)

        def wgt_body(g, _):
            sl = pl.ds(g * _L, _L)
            base_row = ((p0 + g * _L) // pts_per_batch) * pix
            xp = ((gx_v[sl] + 1.0) * img_w - 1.0) * 0.5
            yp = ((gy_v[sl] + 1.0) * img_h - 1.0) * 0.5
            xti = xp.astype(jnp.int32)
            x0 = jnp.where(xti.astype(jnp.float32) > xp, xti - 1, xti)
            yti = yp.astype(jnp.int32)
            y0 = jnp.where(yti.astype(jnp.float32) > yp, yti - 1, yti)
            x0f = x0.astype(jnp.float32)
            y0f = y0.astype(jnp.float32)
            ex = (x0f + 1.0) - xp
            ey = (y0f + 1.0) - yp
            sx = xp - x0f
            sy = yp - y0f
            zero = jnp.zeros_like(ex)
            vx0 = x0 >= 0
            vx1 = x0 <= img_w - 2
            vy0 = y0 >= 0
            vy1 = y0 <= img_h - 2
            wa = jnp.where(vx0 & vy0, ex * ey, zero)
            wb = jnp.where(vx0 & vy1, ex * sy, zero)
            wc = jnp.where(vx1 & vy0, sx * ey, zero)
            wd = jnp.where(vx1 & vy1, sx * sy, zero)
            x0c = jnp.maximum(x0, 0)
            x1c = jnp.minimum(x0 + 1, img_w - 1)
            y0c = jnp.maximum(y0, 0)
            y1c = jnp.minimum(y0 + 1, img_h - 1)
            ra = base_row + y0c * img_w + x0c
            rb = base_row + y1c * img_w + x0c
            rc = base_row + y0c * img_w + x1c
            rd = base_row + y1c * img_w + x1c
            wa_v[sl] = wa
            wb_v[sl] = wb
            wc_v[sl] = wc
            wd_v[sl] = wd
            ib = g * (4 * _L)
            idx_v[pl.ds(ib, _L)] = ra
            idx_v[pl.ds(ib + _L, _L)] = rb
            idx_v[pl.ds(ib + 2 * _L, _L)] = rc
            idx_v[pl.ds(ib + 3 * _L, _L)] = rd
            return 0

        lax.fori_loop(0, n_grp, wgt_body, 0)

        cmax = chw + 1

        def start_g(ch, rbuf, s):
            pltpu.async_copy(
                xt_hbm.at[idx_v.at[pl.ds(ch * nidx, nidx)]], rbuf, s)

        def wait_g(rbuf, s):
            pltpu.make_async_copy(
                xt_hbm.at[idx_v.at[pl.ds(0, nidx)]], rbuf, s).wait()

        lo_blk = n_words // _L
        hi_blk = (n_chan - n_words) // _L
        mask_hi = jnp.int32(-65536)

        def compute_out(ch, rbuf):
            def pt_body(j, _):
                p = ch * cpts + j
                was = wa_v[pl.ds(p, _L)][0]
                wbs = wb_v[pl.ds(p, _L)][0]
                wcs = wc_v[pl.ds(p, _L)][0]
                wds = wd_v[pl.ds(p, _L)][0]
                r0 = (4 * _L) * (j // _L) + (j % _L)
                for k in range(lo_blk):
                    cs = pl.ds(k * _L, _L)
                    va = rbuf[r0, cs]
                    vb = rbuf[r0 + _L, cs]
                    vc = rbuf[r0 + 2 * _L, cs]
                    vd = rbuf[r0 + 3 * _L, cs]
                    acc = (was * lax.bitcast_convert_type(va << 16, jnp.float32)
                           + wbs * lax.bitcast_convert_type(vb << 16, jnp.float32)
                           + wcs * lax.bitcast_convert_type(vc << 16, jnp.float32)
                           + wds * lax.bitcast_convert_type(vd << 16, jnp.float32))
                    out_v[j, cs] = acc
                    if k < hi_blk:
                        acch = (was * lax.bitcast_convert_type(va & mask_hi, jnp.float32)
                                + wbs * lax.bitcast_convert_type(vb & mask_hi, jnp.float32)
                                + wcs * lax.bitcast_convert_type(vc & mask_hi, jnp.float32)
                                + wds * lax.bitcast_convert_type(vd & mask_hi, jnp.float32))
                        out_v[j, pl.ds(n_words + k * _L, _L)] = acch
                return 0

            lax.fori_loop(0, cpts, pt_body, 0)

            @pl.when(ch < n_my)
            def _():
                gr = (start_c + ch) * cpts
                bat = gr // pts_per_batch
                pltpu.sync_copy(
                    out_v,
                    out_hbm.at[bat, pl.ds(gr - bat * pts_per_batch, cpts)])

        if cmax % 2 == 1:
            start_g(0, rows_v0, sem0)

            def pair_body(i, _):
                ch0 = 2 * i
                wait_g(rows_v0, sem0)
                start_g(ch0 + 1, rows_v1, sem1)
                compute_out(ch0, rows_v0)
                wait_g(rows_v1, sem1)
                start_g(ch0 + 2, rows_v0, sem0)
                compute_out(ch0 + 1, rows_v1)
                return 0

            lax.fori_loop(0, (cmax - 1) // 2, pair_body, 0)
            wait_g(rows_v0, sem0)
            compute_out(cmax - 1, rows_v0)
        else:
            def chunk_body(ch, _):
                start_g(ch, rows_v0, sem0)
                wait_g(rows_v0, sem0)
                compute_out(ch, rows_v0)
                return 0

            lax.fori_loop(0, cmax, chunk_body, 0)

    return body(xt, gx, gy)


def kernel(x, pos, H, W):
    b, c, img_h, img_w = x.shape
    n = pos.shape[1]
    hw = img_h * img_w
    u16 = jax.lax.bitcast_convert_type(x.astype(jnp.bfloat16), jnp.uint16)
    u = u16.astype(jnp.uint32)
    lo = u[:, :128]
    hi = jnp.pad(u[:, 128:], ((0, 0), (0, 256 - c), (0, 0), (0, 0)))
    words = jax.lax.bitcast_convert_type(lo | (hi << 16), jnp.int32)
    xt = jnp.transpose(words.reshape(b, 128, hw), (0, 2, 1))
    xt = xt.reshape(b * hw, 128)
    scale = jnp.array([W - 1, H - 1], dtype=x.dtype)
    grid = 2.0 * (pos / scale) - 1.0
    n_pts = b * n
    cpts = 32
    total_ch = n_pts // cpts
    chw = total_ch // _NW
    pad = (chw + 1) * cpts + (total_ch - 1) * 0
    gx = jnp.pad(grid[..., 0].reshape(-1), (0, pad))
    gy = jnp.pad(grid[..., 1].reshape(-1), (0, pad))
    return _sc_interp(xt, gx, gy, img_h=img_h, img_w=img_w, n_chan=c,
                      n_pts=n_pts, pts_per_batch=n)


# speedup vs baseline: 1.6400x; 1.1670x over previous
import functools

import jax
import jax.numpy as jnp
from jax import lax
from jax.experimental import pallas as pl
from jax.experimental.pallas import tpu as pltpu
from jax.experimental.pallas import tpu_sc as plsc

_NC = 2
_NS = 16
_NW = _NC * _NS
_L = 16


def _sc_interp(xt, gx, gy, *, img_h, img_w, n_chan, n_pts, pts_per_batch):
    n_words = xt.shape[1]
    cpts = 32
    nidx = 4 * cpts
    total_ch = n_pts // cpts
    chw, chr = divmod(total_ch, _NW)
    ppw_buf = (chw + 1) * cpts
    n_grp = ppw_buf // _L
    cvecs = n_chan // _L
    pix = img_h * img_w

    mesh = plsc.VectorSubcoreMesh(core_axis_name="c", subcore_axis_name="s",
                                  num_cores=_NC, num_subcores=_NS)

    n_bat = n_pts // pts_per_batch

    @functools.partial(
        pl.kernel,
        out_type=jax.ShapeDtypeStruct((n_bat, pts_per_batch, n_chan),
                                      jnp.float32),
        mesh=mesh,
        scratch_types=[
            pltpu.VMEM((ppw_buf,), jnp.float32),
            pltpu.VMEM((ppw_buf,), jnp.float32),
            pltpu.VMEM((ppw_buf + _L,), jnp.float32),
            pltpu.VMEM((ppw_buf + _L,), jnp.float32),
            pltpu.VMEM((ppw_buf + _L,), jnp.float32),
            pltpu.VMEM((ppw_buf + _L,), jnp.float32),
            pltpu.VMEM((4 * ppw_buf,), jnp.int32),
            pltpu.VMEM((nidx, n_words), jnp.int32),
            pltpu.VMEM((nidx, n_words), jnp.int32),
            pltpu.VMEM((cpts, n_chan), jnp.float32),
            pltpu.SemaphoreType.DMA,
            pltpu.SemaphoreType.DMA,
        ],
    )
    def body(xt_hbm, gx_hbm, gy_hbm, out_hbm,
             gx_v, gy_v, wa_v, wb_v, wc_v, wd_v, idx_v, rows_v0, rows_v1,
             out_v, sem0, sem1):
        wid = lax.axis_index("s") * _NC + lax.axis_index("c")
        start_c = chw * wid + jnp.minimum(wid, chr)
        n_my = jnp.where(wid < chr, chw + 1, chw)
        p0 = start_c * cpts

        pltpu.sync_copy(gx_hbm.at[pl.ds(p0, ppw_buf)], gx_v)
        pltpu.sync_copy(gy_hbm.at[pl.ds(p0, ppw_buf)], gy_v)

        def wgt_body(g, _):
            sl = pl.ds(g * _L, _L)
            base_row = ((p0 + g * _L) // pts_per_batch) * pix
            xp = ((gx_v[sl] + 1.0) * img_w - 1.0) * 0.5
            yp = ((gy_v[sl] + 1.0) * img_h - 1.0) * 0.5
            xti = xp.astype(jnp.int32)
            x0 = jnp.where(xti.astype(jnp.float32) > xp, xti - 1, xti)
            yti = yp.astype(jnp.int32)
            y0 = jnp.where(yti.astype(jnp.float32) > yp, yti - 1, yti)
            x0f = x0.astype(jnp.float32)
            y0f = y0.astype(jnp.float32)
            ex = (x0f + 1.0) - xp
            ey = (y0f + 1.0) - yp
            sx = xp - x0f
            sy = yp - y0f
            zero = jnp.zeros_like(ex)
            vx0 = x0 >= 0
            vx1 = x0 <= img_w - 2
            vy0 = y0 >= 0
            vy1 = y0 <= img_h - 2
            wa = jnp.where(vx0 & vy0, ex * ey, zero)
            wb = jnp.where(vx0 & vy1, ex * sy, zero)
            wc = jnp.where(vx1 & vy0, sx * ey, zero)
            wd = jnp.where(vx1 & vy1, sx * sy, zero)
            x0c = jnp.maximum(x0, 0)
            x1c = jnp.minimum(x0 + 1, img_w - 1)
            y0c = jnp.maximum(y0, 0)
            y1c = jnp.minimum(y0 + 1, img_h - 1)
            ra = base_row + y0c * img_w + x0c
            rb = base_row + y1c * img_w + x0c
            rc = base_row + y0c * img_w + x1c
            rd = base_row + y1c * img_w + x1c
            wa_v[sl] = wa
            wb_v[sl] = wb
            wc_v[sl] = wc
            wd_v[sl] = wd
            ib = g * (4 * _L)
            idx_v[pl.ds(ib, _L)] = ra
            idx_v[pl.ds(ib + _L, _L)] = rb
            idx_v[pl.ds(ib + 2 * _L, _L)] = rc
            idx_v[pl.ds(ib + 3 * _L, _L)] = rd
            return 0

        lax.fori_loop(0, n_grp, wgt_body, 0)

        cmax = chw + 1

        def start_g(ch, rbuf, s):
            pltpu.async_copy(
                xt_hbm.at[idx_v.at[pl.ds(ch * nidx, nidx)]], rbuf, s)

        def wait_g(rbuf, s):
            pltpu.make_async_copy(
                xt_hbm.at[idx_v.at[pl.ds(0, nidx)]], rbuf, s).wait()

        lo_blk = n_words // _L
        hi_blk = (n_chan - n_words) // _L
        mask_hi = jnp.int32(-65536)

        def compute_out(ch, rbuf):
            def pt_body(j, _):
                p = ch * cpts + j
                was = wa_v[pl.ds(p, _L)][0]
                wbs = wb_v[pl.ds(p, _L)][0]
                wcs = wc_v[pl.ds(p, _L)][0]
                wds = wd_v[pl.ds(p, _L)][0]
                r0 = (4 * _L) * (j // _L) + (j % _L)
                for k in range(lo_blk):
                    cs = pl.ds(k * _L, _L)
                    va = rbuf[r0, cs]
                    vb = rbuf[r0 + _L, cs]
                    vc = rbuf[r0 + 2 * _L, cs]
                    vd = rbuf[r0 + 3 * _L, cs]
                    acc = (was * lax.bitcast_convert_type(va << 16, jnp.float32)
                           + wbs * lax.bitcast_convert_type(vb << 16, jnp.float32)
                           + wcs * lax.bitcast_convert_type(vc << 16, jnp.float32)
                           + wds * lax.bitcast_convert_type(vd << 16, jnp.float32))
                    out_v[j, cs] = acc
                    if k < hi_blk:
                        acch = (was * lax.bitcast_convert_type(va & mask_hi, jnp.float32)
                                + wbs * lax.bitcast_convert_type(vb & mask_hi, jnp.float32)
                                + wcs * lax.bitcast_convert_type(vc & mask_hi, jnp.float32)
                                + wds * lax.bitcast_convert_type(vd & mask_hi, jnp.float32))
                        out_v[j, pl.ds(n_words + k * _L, _L)] = acch
                return 0

            @plsc.parallel_loop(0, cpts, unroll=2)
            def _(j):
                pt_body(j, 0)

            @pl.when(ch < n_my)
            def _():
                gr = (start_c + ch) * cpts
                bat = gr // pts_per_batch
                pltpu.sync_copy(
                    out_v,
                    out_hbm.at[bat, pl.ds(gr - bat * pts_per_batch, cpts)])

        if cmax % 2 == 1:
            start_g(0, rows_v0, sem0)

            def pair_body(i, _):
                ch0 = 2 * i
                wait_g(rows_v0, sem0)
                start_g(ch0 + 1, rows_v1, sem1)
                compute_out(ch0, rows_v0)
                wait_g(rows_v1, sem1)
                start_g(ch0 + 2, rows_v0, sem0)
                compute_out(ch0 + 1, rows_v1)
                return 0

            lax.fori_loop(0, (cmax - 1) // 2, pair_body, 0)
            wait_g(rows_v0, sem0)
            compute_out(cmax - 1, rows_v0)
        else:
            def chunk_body(ch, _):
                start_g(ch, rows_v0, sem0)
                wait_g(rows_v0, sem0)
                compute_out(ch, rows_v0)
                return 0

            lax.fori_loop(0, cmax, chunk_body, 0)

    return body(xt, gx, gy)


def kernel(x, pos, H, W):
    b, c, img_h, img_w = x.shape
    n = pos.shape[1]
    hw = img_h * img_w
    u16 = jax.lax.bitcast_convert_type(x.astype(jnp.bfloat16), jnp.uint16)
    u = u16.astype(jnp.uint32)
    lo = u[:, :128]
    hi = jnp.pad(u[:, 128:], ((0, 0), (0, 256 - c), (0, 0), (0, 0)))
    words = jax.lax.bitcast_convert_type(lo | (hi << 16), jnp.int32)
    xt = jnp.transpose(words.reshape(b, 128, hw), (0, 2, 1))
    xt = xt.reshape(b * hw, 128)
    scale = jnp.array([W - 1, H - 1], dtype=x.dtype)
    grid = 2.0 * (pos / scale) - 1.0
    n_pts = b * n
    cpts = 32
    total_ch = n_pts // cpts
    chw = total_ch // _NW
    pad = (chw + 1) * cpts + (total_ch - 1) * 0
    gx = jnp.pad(grid[..., 0].reshape(-1), (0, pad))
    gy = jnp.pad(grid[..., 1].reshape(-1), (0, pad))
    return _sc_interp(xt, gx, gy, img_h=img_h, img_w=img_w, n_chan=c,
                      n_pts=n_pts, pts_per_batch=n)
